# baseline (speedup 1.0000x reference)
import functools

import jax
import jax.numpy as jnp
from jax import lax
from jax.experimental import pallas as pl
from jax.experimental.pallas import tpu as pltpu
from jax.experimental.pallas import tpu_sc as plsc

_D = 64

_mesh = plsc.ScalarSubcoreMesh(axis_name="c", num_cores=1)


def kernel(x, W):
    xs = jnp.asarray(x, jnp.int32).reshape(())

    @functools.partial(
        pl.kernel,
        out_type=jax.ShapeDtypeStruct((1, _D), jnp.float32),
        mesh=_mesh,
        compiler_params=pltpu.CompilerParams(use_tc_tiling_on_sc=False),
    )
    def _gather_row(table_hbm, out_hbm):
        pltpu.sync_copy(table_hbm.at[pl.ds(xs, 1)], out_hbm)

    return _gather_row(W)


# speedup vs baseline: 1.0186x; 1.0186x over previous
import functools

import jax
import jax.numpy as jnp
from jax import lax
from jax.experimental import pallas as pl
from jax.experimental.pallas import tpu as pltpu
from jax.experimental.pallas import tpu_sc as plsc

_D = 64

_mesh = plsc.ScalarSubcoreMesh(axis_name="c", num_cores=1)


def kernel(x, W):
    xs = jnp.asarray(x, jnp.int32).reshape(())

    @functools.partial(
        pl.kernel,
        out_type=jax.ShapeDtypeStruct((1, _D), jnp.float32),
        mesh=_mesh,
        compiler_params=pltpu.CompilerParams(skip_device_barrier=True),
    )
    def _gather_row(table_hbm, out_hbm):
        pltpu.sync_copy(table_hbm.at[pl.ds(xs, 1)], out_hbm)

    return _gather_row(W)


# speedup vs baseline: 1.0357x; 1.0168x over previous
import functools

import jax
import jax.numpy as jnp
from jax import lax
from jax.experimental import pallas as pl
from jax.experimental.pallas import tpu as pltpu
from jax.experimental.pallas import tpu_sc as plsc

_D = 64

_mesh = plsc.ScalarSubcoreMesh(axis_name="c", num_cores=1)


def kernel(x, W):
    xs = jnp.asarray(x, jnp.int32).reshape(())

    @functools.partial(
        pl.kernel,
        out_type=jax.ShapeDtypeStruct((1, _D), jnp.float32),
        mesh=_mesh,
    )
    def _gather_row(table_hbm, out_hbm):
        pltpu.sync_copy(table_hbm.at[pl.ds(xs, 1)], out_hbm)

    return _gather_row(W)
